# Initial kernel scaffold; baseline (speedup 1.0000x reference)
import jax
import jax.numpy as jnp
from jax.experimental import pallas as pl


def kernel(coords, feats, W, b):
    raise NotImplementedError("write your pallas kernel here")


# speedup vs baseline: 3.7084x; 3.7084x over previous
import jax
import jax.numpy as jnp
from jax.experimental import pallas as pl
from jax.experimental.pallas import tpu as pltpu

G = 256
NIN = 128
NOUT = 128
GO2 = 257


def _conv_body(fc_ref, fp_ref, cc_ref, cp_ref, w05_ref, w68_ref, b_ref, out_ref):
    u = pl.program_id(0)
    fvalid = (u < G).astype(jnp.float32)
    pvalid = (u > 0).astype(jnp.float32)
    fc = fc_ref[0] * fvalid
    fp = fp_ref[0] * pvalid
    cc = cc_ref[0] * fvalid
    cp = cp_ref[0] * pvalid
    P = jnp.dot(fc, w05_ref[...], preferred_element_type=jnp.float32)
    Q = jnp.dot(fp, w68_ref[...], preferred_element_type=jnp.float32)

    z = jnp.zeros((1, NOUT), jnp.float32)
    ext = lambda x: jnp.concatenate([x, z], axis=0)
    sh = lambda x: jnp.concatenate([z, x], axis=0)
    P0, P1, P2 = P[:, 0:128], P[:, 128:256], P[:, 256:384]
    P3, P4, P5 = P[:, 384:512], P[:, 512:640], P[:, 640:768]
    Q6, Q7, Q8 = Q[:, 0:128], Q[:, 128:256], Q[:, 256:384]
    A0 = ext(P0) + sh(P2) + ext(Q6) + sh(Q8)
    A1 = ext(P1) + ext(Q7)
    B0 = ext(P3) + sh(P5)
    B1 = ext(P4)

    zc = jnp.zeros((1, 1), jnp.float32)
    extc = lambda x: jnp.concatenate([x, zc], axis=0)
    shc = lambda x: jnp.concatenate([zc, x], axis=0)
    cA0 = extc(cc) + shc(cc) + extc(cp) + shc(cp)
    cA1 = extc(cc) + extc(cp)
    cB0 = extc(cc) + shc(cc)
    cB1 = extc(cc)

    b = b_ref[...]
    fin = lambda v, c: jnp.where(c > 0, jnp.maximum(v + b, 0.0), 0.0)
    out_ref[0, 0, :, 0, :] = fin(A0, cA0)
    out_ref[0, 0, :, 1, :] = fin(A1, cA1)
    out_ref[0, 1, :, 0, :] = fin(B0, cB0)
    out_ref[0, 1, :, 1, :] = fin(B1, cB1)


def _parity_conv(F3, C3, W05, W68, b2, interpret=False):
    return pl.pallas_call(
        _conv_body,
        grid=(GO2,),
        in_specs=[
            pl.BlockSpec((1, G, NIN), lambda u: (jnp.minimum(u, G - 1), 0, 0)),
            pl.BlockSpec((1, G, NIN), lambda u: (jnp.maximum(u - 1, 0), 0, 0)),
            pl.BlockSpec((1, G, 1), lambda u: (jnp.minimum(u, G - 1), 0, 0)),
            pl.BlockSpec((1, G, 1), lambda u: (jnp.maximum(u - 1, 0), 0, 0)),
            pl.BlockSpec((NIN, 6 * NOUT), lambda u: (0, 0)),
            pl.BlockSpec((NIN, 3 * NOUT), lambda u: (0, 0)),
            pl.BlockSpec((1, NOUT), lambda u: (0, 0)),
        ],
        out_specs=pl.BlockSpec((1, 2, GO2, 2, NOUT), lambda u: (u, 0, 0, 0, 0)),
        out_shape=jax.ShapeDtypeStruct((GO2, 2, GO2, 2, NOUT), jnp.float32),
        interpret=interpret,
    )(F3, F3, C3, C3, W05, W68, b2)


def _scatter_dense_jnp(coords, feats):
    cell = coords // 2
    lin = cell[:, 0] * G + cell[:, 1]
    F = jnp.zeros((G * G, NIN), jnp.float32).at[lin].add(feats)
    C = jnp.zeros((G * G,), jnp.float32).at[lin].add(1.0)
    return F, C


def kernel(coords, feats, W, b, interpret=False):
    F, C = _scatter_dense_jnp(coords, feats)
    F3 = F.reshape(G, G, NIN)
    C3 = C.reshape(G, G, 1)
    W05 = jnp.concatenate([W[k] for k in range(6)], axis=1)
    W68 = jnp.concatenate([W[k] for k in range(6, 9)], axis=1)
    b2 = b.reshape(1, NOUT)
    out5 = _parity_conv(F3, C3, W05, W68, b2, interpret=interpret)
    return out5.reshape((2 * GO2) * (2 * GO2), NOUT)
